# baseline (speedup 1.0000x reference)
import functools

import numpy as np
import jax
import jax.numpy as jnp
from jax import lax
from jax.experimental import pallas as pl
from jax.experimental.pallas import tpu as pltpu
from jax.experimental.pallas import tpu_sc as plsc

N_FEATURES = 26
INPUT_DIM = 101
OUT_DIM = 3
BATCH = 16384
ROW = N_FEATURES * OUT_DIM
NUM_WORKERS = 32
ROWS_PER_W = BATCH // NUM_WORKERS
BLOCK_ROWS = 8
BLOCK_ELEMS = BLOCK_ROWS * ROW
VECS_PER_BLOCK = BLOCK_ELEMS // 16
BLOCKS_PER_W = ROWS_PER_W // BLOCK_ROWS
IDX_PER_W = ROWS_PER_W * N_FEATURES
OUT_PER_W = ROWS_PER_W * ROW
TAB_SIZE = N_FEATURES * INPUT_DIM * OUT_DIM

_pos = np.arange(BLOCK_ELEMS)
_brow = _pos // ROW
_col = _pos % ROW
_feat = _col // OUT_DIM
_comp = _col % OUT_DIM
_TMPL_IDX = (_brow * N_FEATURES + _feat).astype(np.int32)
_TMPL_TAB = (_feat * (INPUT_DIM * OUT_DIM) + _comp).astype(np.int32)


def _sc_embed(idx_flat, tab_flat, tmpl_idx, tmpl_tab):
    mesh = plsc.VectorSubcoreMesh(core_axis_name="c", subcore_axis_name="s")

    @functools.partial(
        pl.kernel,
        mesh=mesh,
        out_type=jax.ShapeDtypeStruct((BATCH * ROW,), jnp.float32),
        compiler_params=pltpu.CompilerParams(needs_layout_passes=False),
        scratch_types=[
            pltpu.VMEM((IDX_PER_W,), jnp.int32),
            pltpu.VMEM((TAB_SIZE,), jnp.float32),
            pltpu.VMEM((BLOCK_ELEMS,), jnp.int32),
            pltpu.VMEM((BLOCK_ELEMS,), jnp.int32),
            pltpu.VMEM((OUT_PER_W,), jnp.float32),
        ],
    )
    def k(idx_hbm, tab_hbm, ta_hbm, tb_hbm, out_hbm, idx_v, tab_v, ta_v, tb_v, out_v):
        wid = lax.axis_index("s") * 2 + lax.axis_index("c")
        pltpu.sync_copy(idx_hbm.at[pl.ds(wid * IDX_PER_W, IDX_PER_W)], idx_v)
        pltpu.sync_copy(tab_hbm, tab_v)
        pltpu.sync_copy(ta_hbm, ta_v)
        pltpu.sync_copy(tb_hbm, tb_v)

        for j in range(VECS_PER_BLOCK):
            a = ta_v[pl.ds(j * 16, 16)]
            t = tb_v[pl.ds(j * 16, 16)]

            @plsc.parallel_loop(0, BLOCKS_PER_W)
            def blk(b, a=a, t=t, j=j):
                iv = plsc.load_gather(idx_v, [a + b * (BLOCK_ROWS * N_FEATURES)])
                e = plsc.load_gather(tab_v, [t + iv * OUT_DIM])
                out_v[pl.ds(b * BLOCK_ELEMS + j * 16, 16)] = e
        pltpu.sync_copy(out_v, out_hbm.at[pl.ds(wid * OUT_PER_W, OUT_PER_W)])

    return k(idx_flat, tab_flat, tmpl_idx, tmpl_tab)


def kernel(inputs, tables):
    out = _sc_embed(inputs.reshape(-1), tables.reshape(-1), _TMPL_IDX, _TMPL_TAB)
    return out.reshape(BATCH, ROW)


# speedup vs baseline: 1.0388x; 1.0388x over previous
import functools

import numpy as np
import jax
import jax.numpy as jnp
from jax import lax
from jax.experimental import pallas as pl
from jax.experimental.pallas import tpu as pltpu
from jax.experimental.pallas import tpu_sc as plsc

N_FEATURES = 26
INPUT_DIM = 101
OUT_DIM = 3
BATCH = 16384
ROW = N_FEATURES * OUT_DIM
NUM_WORKERS = 32
ROWS_PER_W = BATCH // NUM_WORKERS
BLOCK_ROWS = 8
BLOCK_ELEMS = BLOCK_ROWS * ROW
VECS_PER_BLOCK = BLOCK_ELEMS // 16
BLOCKS_PER_W = ROWS_PER_W // BLOCK_ROWS
IDX_PER_W = ROWS_PER_W * N_FEATURES
OUT_PER_W = ROWS_PER_W * ROW
TAB_SIZE = N_FEATURES * INPUT_DIM * OUT_DIM

_pos = np.arange(BLOCK_ELEMS)
_brow = _pos // ROW
_col = _pos % ROW
_feat = _col // OUT_DIM
_comp = _col % OUT_DIM
_TMPL = ((_brow * N_FEATURES + _feat)
         | ((_feat * (INPUT_DIM * OUT_DIM) + _comp) << 16)).astype(np.int32)


def _sc_embed(idx_flat, tab_flat, tmpl):
    mesh = plsc.VectorSubcoreMesh(core_axis_name="c", subcore_axis_name="s")

    @functools.partial(
        pl.kernel,
        mesh=mesh,
        out_type=jax.ShapeDtypeStruct((BATCH * ROW,), jnp.float32),
        compiler_params=pltpu.CompilerParams(needs_layout_passes=False),
        scratch_types=[
            pltpu.VMEM((IDX_PER_W,), jnp.int32),
            pltpu.VMEM((TAB_SIZE,), jnp.float32),
            pltpu.VMEM((BLOCK_ELEMS,), jnp.int32),
            pltpu.VMEM((OUT_PER_W,), jnp.float32),
        ],
    )
    def k(idx_hbm, tab_hbm, tp_hbm, out_hbm, idx_v, tab_v, tp_v, out_v):
        wid = lax.axis_index("s") * 2 + lax.axis_index("c")
        pltpu.sync_copy(idx_hbm.at[pl.ds(wid * IDX_PER_W, IDX_PER_W)], idx_v)
        pltpu.sync_copy(tab_hbm, tab_v)
        pltpu.sync_copy(tp_hbm, tp_v)

        @plsc.parallel_loop(0, BLOCKS_PER_W, unroll=2)
        def blk(b):
            ibase = b * (BLOCK_ROWS * N_FEATURES)
            obase = b * BLOCK_ELEMS
            for j in range(VECS_PER_BLOCK):
                p = tp_v[pl.ds(j * 16, 16)]
                a = p & 0xFFFF
                t = p >> 16
                iv = plsc.load_gather(idx_v, [a + ibase])
                e = plsc.load_gather(tab_v, [t + iv * OUT_DIM])
                out_v[pl.ds(obase + j * 16, 16)] = e

        pltpu.sync_copy(out_v, out_hbm.at[pl.ds(wid * OUT_PER_W, OUT_PER_W)])

    return k(idx_flat, tab_flat, tmpl)


def kernel(inputs, tables):
    out = _sc_embed(inputs.reshape(-1), tables.reshape(-1), _TMPL)
    return out.reshape(BATCH, ROW)
